# baseline (speedup 1.0000x reference)
import functools

import jax
import jax.numpy as jnp
from jax import lax
from jax.experimental import pallas as pl
from jax.experimental.pallas import tpu as pltpu
from jax.experimental.pallas import tpu_sc as plsc

P = 16
D = 32
M = 8192
TBLK = 128


def _cbfull_body(cb_ref, dec_w_ref, dec_b_ref, out_ref, cbn2_ref):
    cb = cb_ref[...]
    out_ref[...] = jnp.dot(cb, dec_w_ref[...],
                           preferred_element_type=jnp.float32) + dec_b_ref[...]
    cbn2_ref[...] = jnp.sum(cb * cb, axis=1).reshape(1, -1)


def _codebook_full(codebook, dec_W, dec_b_row, F):
    nblk = 8
    rb = M // nblk
    return pl.pallas_call(
        _cbfull_body,
        grid=(nblk,),
        in_specs=[pl.BlockSpec((rb, D), lambda i: (i, 0)),
                  pl.BlockSpec((D, F), lambda i: (0, 0)),
                  pl.BlockSpec((1, F), lambda i: (0, 0))],
        out_specs=[pl.BlockSpec((rb, F), lambda i: (i, 0)),
                   pl.BlockSpec((1, rb), lambda i: (0, i))],
        out_shape=[jax.ShapeDtypeStruct((M, F), jnp.float32),
                   jax.ShapeDtypeStruct((1, M), jnp.float32)],
    )(codebook, dec_W, dec_b_row)


def _make_patchify_sc(B, C, H, W, F, row0, nrow):
    wp = W // P
    hp = H // P
    info = plsc.get_sparse_core_info()
    NW = info.num_cores * info.num_subcores
    rows_per_w = nrow // NW
    mesh = plsc.VectorSubcoreMesh(core_axis_name="c", subcore_axis_name="s")

    @functools.partial(
        pl.kernel, mesh=mesh,
        out_type=jax.ShapeDtypeStruct((nrow * wp, F), jnp.float32),
        scratch_types=[pltpu.VMEM((2, C, P, W), jnp.float32),
                       pltpu.VMEM((2, wp, F), jnp.float32),
                       pltpu.SemaphoreType.DMA,
                       pltpu.SemaphoreType.DMA,
                       pltpu.SemaphoreType.DMA,
                       pltpu.SemaphoreType.DMA],
    )
    def patchify(x_hbm, patches_hbm, slab_v, patch_v, s0, s1, w0, w1):
        wid = lax.axis_index("s") * info.num_cores + lax.axis_index("c")
        rsem = (s0, s1)
        wsem = (w0, w1)

        def fire_reads(k, buf):
            row = row0 + wid * rows_per_w + k
            b = row // hp
            i = row % hp
            return [pltpu.async_copy(x_hbm.at[b, c, pl.ds(i * P, P)],
                                     slab_v.at[buf, c], rsem[buf])
                    for c in range(C)]

        reads = fire_reads(0, 0)
        writes = [None, None]
        for k in range(rows_per_w):
            cur = k & 1
            nxt = 1 - cur
            if k + 1 < rows_per_w:
                nreads = fire_reads(k + 1, nxt)
            for d in reads:
                d.wait()
            if writes[cur] is not None:
                writes[cur].wait()

            def rearrange(j, _):
                for c in range(C):
                    for pr in range(P):
                        patch_v[cur, j, pl.ds((c * P + pr) * P, P)] = (
                            slab_v[cur, c, pr, pl.ds(j * P, P)])
                return 0

            lax.fori_loop(0, wp, rearrange, 0)
            lrow = wid * rows_per_w + k
            writes[cur] = pltpu.async_copy(
                patch_v.at[cur], patches_hbm.at[pl.ds(lrow * wp, wp)],
                wsem[cur])
            if k + 1 < rows_per_w:
                reads = nreads
        for d in writes:
            if d is not None:
                d.wait()

    return patchify


def _assign_body(nblk, p_ref, enc_w_ref, enc_b_ref, cbt_ref, cbn2_ref,
                 ind_ref, vq_ref):
    s = pl.program_id(0)
    lat = jnp.dot(p_ref[...], enc_w_ref[...],
                  preferred_element_type=jnp.float32) + enc_b_ref[...]
    scores = jnp.dot(lat, cbt_ref[...], preferred_element_type=jnp.float32)
    best = jnp.max(scores, axis=1)

    msk = scores == best[:, None]
    iota = jax.lax.broadcasted_iota(jnp.int32, (TBLK, M), 1)
    ind = jnp.min(jnp.where(msk, iota, M), axis=1).astype(jnp.int32)
    qn2 = jnp.sum(jnp.where(msk, cbn2_ref[...], 0.0), axis=1)
    ln2 = jnp.sum(lat * lat, axis=1)
    cos = best / jnp.maximum(jnp.sqrt(ln2) * jnp.sqrt(qn2), 1e-8)
    vq_blk = jnp.sum(1.0 - cos)

    ind_ref[...] = ind.reshape(1, 1, TBLK)

    @pl.when(s == 0)
    def _():
        vq_ref[0, 0] = 0.0

    vq_ref[0, 0] += vq_blk


def _assign(patches, enc_W, enc_b_row, codebook_T, cbn2, N, F):
    nblk = N // TBLK
    ind, vq = pl.pallas_call(
        functools.partial(_assign_body, nblk),
        grid=(nblk,),
        in_specs=[pl.BlockSpec((TBLK, F), lambda s: (s, 0)),
                  pl.BlockSpec((F, D), lambda s: (0, 0)),
                  pl.BlockSpec((1, D), lambda s: (0, 0)),
                  pl.BlockSpec((D, M), lambda s: (0, 0)),
                  pl.BlockSpec((1, M), lambda s: (0, 0))],
        out_specs=[pl.BlockSpec((1, 1, TBLK), lambda s: (s, 0, 0)),
                   pl.BlockSpec((1, 1), lambda s: (0, 0),
                                memory_space=pltpu.SMEM)],
        out_shape=[jax.ShapeDtypeStruct((nblk, 1, TBLK), jnp.int32),
                   jax.ShapeDtypeStruct((1, 1), jnp.float32)],
    )(patches, enc_W, enc_b_row, codebook_T, cbn2)
    return ind.reshape(N), vq


def _make_decode_sc(B, C, H, W, F):
    wp = W // P
    hp = H // P
    nrow = B * hp
    info = plsc.get_sparse_core_info()
    NW = info.num_cores * info.num_subcores
    rows_per_w = nrow // NW
    mesh = plsc.VectorSubcoreMesh(core_axis_name="c", subcore_axis_name="s")

    @functools.partial(
        pl.kernel, mesh=mesh,
        out_type=jax.ShapeDtypeStruct((B, C, H, W), jnp.float32),
        scratch_types=[pltpu.VMEM((2, wp), jnp.int32),
                       pltpu.VMEM((2, wp, F), jnp.float32),
                       pltpu.VMEM((2, C, P, W), jnp.float32),
                       pltpu.SemaphoreType.DMA,
                       pltpu.SemaphoreType.DMA,
                       pltpu.SemaphoreType.DMA,
                       pltpu.SemaphoreType.DMA],
    )
    def decode(cbfull_hbm, ind_hbm, out_hbm, idx_v, rows_v, slab_v,
               g0, g1, w0, w1):
        wid = lax.axis_index("s") * info.num_cores + lax.axis_index("c")
        gsem = (g0, g1)
        wsem = (w0, w1)

        def fire_gather(k, buf):
            row = wid * rows_per_w + k
            pltpu.sync_copy(ind_hbm.at[pl.ds(row * wp, wp)], idx_v.at[buf])
            return pltpu.async_copy(cbfull_hbm.at[idx_v.at[buf]],
                                    rows_v.at[buf], gsem[buf])

        gd = fire_gather(0, 0)
        writes = [None, None]
        for k in range(rows_per_w):
            cur = k & 1
            nxt = 1 - cur
            if k + 1 < rows_per_w:
                ngd = fire_gather(k + 1, nxt)
            gd.wait()
            if writes[cur] is not None:
                for d in writes[cur]:
                    d.wait()

            def rearrange(j, _):
                for c in range(C):
                    for pr in range(P):
                        slab_v[cur, c, pr, pl.ds(j * P, P)] = (
                            rows_v[cur, j, pl.ds((c * P + pr) * P, P)])
                return 0

            lax.fori_loop(0, wp, rearrange, 0)
            row = wid * rows_per_w + k
            b = row // hp
            i = row % hp
            writes[cur] = [pltpu.async_copy(
                slab_v.at[cur, c], out_hbm.at[b, c, pl.ds(i * P, P)],
                wsem[cur]) for c in range(C)]
            if k + 1 < rows_per_w:
                gd = ngd
        for ds_ in writes:
            if ds_ is not None:
                for d in ds_:
                    d.wait()

    return decode


def _rec_body(nb, s_ref, t_ref, vq_ref, rec_ref, loss_ref):
    b = pl.program_id(0)
    diff = s_ref[...] - t_ref[...]
    blk = jnp.sum(diff * diff)

    @pl.when(b == 0)
    def _():
        rec_ref[0, 0] = 0.0

    rec_ref[0, 0] += blk

    @pl.when(b == nb - 1)
    def _():
        loss_ref[0, 0] = rec_ref[0, 0] + 0.001 * vq_ref[0, 0]


def _rec_loss(sample, target, vq, B, C, H, W):
    return pl.pallas_call(
        functools.partial(_rec_body, B),
        grid=(B,),
        in_specs=[pl.BlockSpec((1, C, H, W), lambda b: (b, 0, 0, 0)),
                  pl.BlockSpec((1, C, H, W), lambda b: (b, 0, 0, 0)),
                  pl.BlockSpec((1, 1), lambda b: (0, 0),
                               memory_space=pltpu.SMEM)],
        out_specs=[pl.BlockSpec((1, 1), lambda b: (0, 0),
                                memory_space=pltpu.SMEM),
                   pl.BlockSpec((1, 1), lambda b: (0, 0),
                                memory_space=pltpu.SMEM)],
        out_shape=[jax.ShapeDtypeStruct((1, 1), jnp.float32),
                   jax.ShapeDtypeStruct((1, 1), jnp.float32)],
    )(sample, target, vq)


def kernel(input, target, enc_W, enc_b, codebook, dec_W, dec_b):
    B, C, H, W = input.shape
    F = C * P * P
    hp = H // P
    nrow = B * hp
    half = nrow // 2

    cbfull, cbn2 = _codebook_full(codebook, dec_W, dec_b.reshape(1, F), F)
    enc_b_row = enc_b.reshape(1, D)
    cbT = codebook.T

    patches_a = _make_patchify_sc(B, C, H, W, F, 0, half)(input)
    patches_b = _make_patchify_sc(B, C, H, W, F, half, half)(input)
    N2 = half * (W // P)
    ind_a, vq_a = _assign(patches_a, enc_W, enc_b_row, cbT, cbn2, N2, F)
    ind_b, vq_b = _assign(patches_b, enc_W, enc_b_row, cbT, cbn2, N2, F)
    vq = vq_a + vq_b
    ind = jnp.concatenate([ind_a, ind_b])
    sample = _make_decode_sc(B, C, H, W, F)(cbfull, ind)
    rec, loss = _rec_loss(sample, target, vq, B, C, H, W)

    return sample, rec[0, 0], vq[0, 0], loss[0, 0]


# speedup vs baseline: 1.0813x; 1.0813x over previous
import functools

import jax
import jax.numpy as jnp
from jax import lax
from jax.experimental import pallas as pl
from jax.experimental.pallas import tpu as pltpu
from jax.experimental.pallas import tpu_sc as plsc

P = 16
D = 32
M = 8192
TBLK = 256


def _cbfull_body(cb_ref, dec_w_ref, dec_b_ref, out_ref, cbn2_ref):
    cb = cb_ref[...]
    out_ref[...] = jnp.dot(cb, dec_w_ref[...],
                           preferred_element_type=jnp.float32) + dec_b_ref[...]
    cbn2_ref[...] = jnp.sum(cb * cb, axis=1).reshape(1, -1)


def _codebook_full(codebook, dec_W, dec_b_row, F):
    nblk = 8
    rb = M // nblk
    return pl.pallas_call(
        _cbfull_body,
        grid=(nblk,),
        in_specs=[pl.BlockSpec((rb, D), lambda i: (i, 0)),
                  pl.BlockSpec((D, F), lambda i: (0, 0)),
                  pl.BlockSpec((1, F), lambda i: (0, 0))],
        out_specs=[pl.BlockSpec((rb, F), lambda i: (i, 0)),
                   pl.BlockSpec((1, rb), lambda i: (0, i))],
        out_shape=[jax.ShapeDtypeStruct((M, F), jnp.float32),
                   jax.ShapeDtypeStruct((1, M), jnp.float32)],
    )(codebook, dec_W, dec_b_row)


def _make_patchify_sc(B, C, H, W, F, row0, nrow):
    wp = W // P
    hp = H // P
    info = plsc.get_sparse_core_info()
    NW = info.num_cores * info.num_subcores
    rows_per_w = nrow // NW
    mesh = plsc.VectorSubcoreMesh(core_axis_name="c", subcore_axis_name="s")

    @functools.partial(
        pl.kernel, mesh=mesh,
        out_type=jax.ShapeDtypeStruct((nrow * wp, F), jnp.float32),
        scratch_types=[pltpu.VMEM((2, C, P, W), jnp.float32),
                       pltpu.VMEM((2, wp, F), jnp.float32),
                       pltpu.SemaphoreType.DMA,
                       pltpu.SemaphoreType.DMA,
                       pltpu.SemaphoreType.DMA,
                       pltpu.SemaphoreType.DMA],
    )
    def patchify(x_hbm, patches_hbm, slab_v, patch_v, s0, s1, w0, w1):
        wid = lax.axis_index("s") * info.num_cores + lax.axis_index("c")
        rsem = (s0, s1)
        wsem = (w0, w1)

        def fire_reads(k, buf):
            row = row0 + wid * rows_per_w + k
            b = row // hp
            i = row % hp
            return [pltpu.async_copy(x_hbm.at[b, c, pl.ds(i * P, P)],
                                     slab_v.at[buf, c], rsem[buf])
                    for c in range(C)]

        reads = fire_reads(0, 0)
        writes = [None, None]
        for k in range(rows_per_w):
            cur = k & 1
            nxt = 1 - cur
            if k + 1 < rows_per_w:
                nreads = fire_reads(k + 1, nxt)
            for d in reads:
                d.wait()
            if writes[cur] is not None:
                writes[cur].wait()

            def rearrange(j, _):
                for c in range(C):
                    for pr in range(P):
                        patch_v[cur, j, pl.ds((c * P + pr) * P, P)] = (
                            slab_v[cur, c, pr, pl.ds(j * P, P)])
                return 0

            lax.fori_loop(0, wp, rearrange, 0)
            lrow = wid * rows_per_w + k
            writes[cur] = pltpu.async_copy(
                patch_v.at[cur], patches_hbm.at[pl.ds(lrow * wp, wp)],
                wsem[cur])
            if k + 1 < rows_per_w:
                reads = nreads
        for d in writes:
            if d is not None:
                d.wait()

    return patchify


def _assign_body(nblk, p_ref, enc_w_ref, enc_b_ref, cbt_ref, cbn2_ref,
                 ind_ref, vq_ref):
    s = pl.program_id(0)
    lat = jnp.dot(p_ref[...], enc_w_ref[...],
                  preferred_element_type=jnp.float32) + enc_b_ref[...]
    scores = jnp.dot(lat, cbt_ref[...], preferred_element_type=jnp.float32)
    best = jnp.max(scores, axis=1)
    ind = jnp.argmax(scores, axis=1).astype(jnp.int32)

    onehot = (jax.lax.broadcasted_iota(jnp.int32, (TBLK, M), 1)
              == ind[:, None]).astype(jnp.float32)
    qn2 = jnp.sum(onehot * cbn2_ref[...], axis=1)
    ln2 = jnp.sum(lat * lat, axis=1)
    cos = best / jnp.maximum(jnp.sqrt(ln2) * jnp.sqrt(qn2), 1e-8)
    vq_blk = jnp.sum(1.0 - cos)

    ind_ref[...] = ind.reshape(1, 1, TBLK)

    @pl.when(s == 0)
    def _():
        vq_ref[0, 0] = 0.0

    vq_ref[0, 0] += vq_blk


def _assign(patches, enc_W, enc_b_row, codebook_T, cbn2, N, F):
    nblk = N // TBLK
    ind, vq = pl.pallas_call(
        functools.partial(_assign_body, nblk),
        grid=(nblk,),
        in_specs=[pl.BlockSpec((TBLK, F), lambda s: (s, 0)),
                  pl.BlockSpec((F, D), lambda s: (0, 0)),
                  pl.BlockSpec((1, D), lambda s: (0, 0)),
                  pl.BlockSpec((D, M), lambda s: (0, 0)),
                  pl.BlockSpec((1, M), lambda s: (0, 0))],
        out_specs=[pl.BlockSpec((1, 1, TBLK), lambda s: (s, 0, 0)),
                   pl.BlockSpec((1, 1), lambda s: (0, 0),
                                memory_space=pltpu.SMEM)],
        out_shape=[jax.ShapeDtypeStruct((nblk, 1, TBLK), jnp.int32),
                   jax.ShapeDtypeStruct((1, 1), jnp.float32)],
    )(patches, enc_W, enc_b_row, codebook_T, cbn2)
    return ind.reshape(N), vq


def _make_decode_sc(B, C, H, W, F):
    wp = W // P
    hp = H // P
    nrow = B * hp
    info = plsc.get_sparse_core_info()
    NW = info.num_cores * info.num_subcores
    rows_per_w = nrow // NW
    mesh = plsc.VectorSubcoreMesh(core_axis_name="c", subcore_axis_name="s")

    @functools.partial(
        pl.kernel, mesh=mesh,
        out_type=jax.ShapeDtypeStruct((B, C, H, W), jnp.float32),
        scratch_types=[pltpu.VMEM((2, wp), jnp.int32),
                       pltpu.VMEM((2, wp, F), jnp.float32),
                       pltpu.VMEM((2, C, P, W), jnp.float32),
                       pltpu.SemaphoreType.DMA,
                       pltpu.SemaphoreType.DMA,
                       pltpu.SemaphoreType.DMA,
                       pltpu.SemaphoreType.DMA],
    )
    def decode(cbfull_hbm, ind_hbm, out_hbm, idx_v, rows_v, slab_v,
               g0, g1, w0, w1):
        wid = lax.axis_index("s") * info.num_cores + lax.axis_index("c")
        gsem = (g0, g1)
        wsem = (w0, w1)

        def fire_gather(k, buf):
            row = wid * rows_per_w + k
            pltpu.sync_copy(ind_hbm.at[pl.ds(row * wp, wp)], idx_v.at[buf])
            return pltpu.async_copy(cbfull_hbm.at[idx_v.at[buf]],
                                    rows_v.at[buf], gsem[buf])

        gd = fire_gather(0, 0)
        writes = [None, None]
        for k in range(rows_per_w):
            cur = k & 1
            nxt = 1 - cur
            if k + 1 < rows_per_w:
                ngd = fire_gather(k + 1, nxt)
            gd.wait()
            if writes[cur] is not None:
                for d in writes[cur]:
                    d.wait()

            def rearrange(j, _):
                for c in range(C):
                    for pr in range(P):
                        slab_v[cur, c, pr, pl.ds(j * P, P)] = (
                            rows_v[cur, j, pl.ds((c * P + pr) * P, P)])
                return 0

            lax.fori_loop(0, wp, rearrange, 0)
            row = wid * rows_per_w + k
            b = row // hp
            i = row % hp
            writes[cur] = [pltpu.async_copy(
                slab_v.at[cur, c], out_hbm.at[b, c, pl.ds(i * P, P)],
                wsem[cur]) for c in range(C)]
            if k + 1 < rows_per_w:
                gd = ngd
        for ds_ in writes:
            if ds_ is not None:
                for d in ds_:
                    d.wait()

    return decode


def _rec_body(nb, s_ref, t_ref, vq_ref, rec_ref, loss_ref):
    b = pl.program_id(0)
    diff = s_ref[...] - t_ref[...]
    blk = jnp.sum(diff * diff)

    @pl.when(b == 0)
    def _():
        rec_ref[0, 0] = 0.0

    rec_ref[0, 0] += blk

    @pl.when(b == nb - 1)
    def _():
        loss_ref[0, 0] = rec_ref[0, 0] + 0.001 * vq_ref[0, 0]


def _rec_loss(sample, target, vq, B, C, H, W):
    return pl.pallas_call(
        functools.partial(_rec_body, B),
        grid=(B,),
        in_specs=[pl.BlockSpec((1, C, H, W), lambda b: (b, 0, 0, 0)),
                  pl.BlockSpec((1, C, H, W), lambda b: (b, 0, 0, 0)),
                  pl.BlockSpec((1, 1), lambda b: (0, 0),
                               memory_space=pltpu.SMEM)],
        out_specs=[pl.BlockSpec((1, 1), lambda b: (0, 0),
                                memory_space=pltpu.SMEM),
                   pl.BlockSpec((1, 1), lambda b: (0, 0),
                                memory_space=pltpu.SMEM)],
        out_shape=[jax.ShapeDtypeStruct((1, 1), jnp.float32),
                   jax.ShapeDtypeStruct((1, 1), jnp.float32)],
    )(sample, target, vq)


def kernel(input, target, enc_W, enc_b, codebook, dec_W, dec_b):
    B, C, H, W = input.shape
    F = C * P * P
    hp = H // P
    nrow = B * hp
    half = nrow // 2

    cbfull, cbn2 = _codebook_full(codebook, dec_W, dec_b.reshape(1, F), F)
    enc_b_row = enc_b.reshape(1, D)
    cbT = codebook.T

    patches_a = _make_patchify_sc(B, C, H, W, F, 0, half)(input)
    patches_b = _make_patchify_sc(B, C, H, W, F, half, half)(input)
    N2 = half * (W // P)
    ind_a, vq_a = _assign(patches_a, enc_W, enc_b_row, cbT, cbn2, N2, F)
    ind_b, vq_b = _assign(patches_b, enc_W, enc_b_row, cbT, cbn2, N2, F)
    vq = vq_a + vq_b
    ind = jnp.concatenate([ind_a, ind_b])
    sample = _make_decode_sc(B, C, H, W, F)(cbfull, ind)
    rec, loss = _rec_loss(sample, target, vq, B, C, H, W)

    return sample, rec[0, 0], vq[0, 0], loss[0, 0]


# speedup vs baseline: 1.0884x; 1.0066x over previous
import functools

import jax
import jax.numpy as jnp
from jax import lax
from jax.experimental import pallas as pl
from jax.experimental.pallas import tpu as pltpu
from jax.experimental.pallas import tpu_sc as plsc

P = 16
D = 32
M = 8192
TBLK = 256


def _cbfull_body(cb_ref, dec_w_ref, dec_b_ref, out_ref, cbn2_ref):
    cb = cb_ref[...]
    out_ref[...] = jnp.dot(cb, dec_w_ref[...],
                           preferred_element_type=jnp.float32) + dec_b_ref[...]
    cbn2_ref[...] = jnp.sum(cb * cb, axis=1).reshape(1, -1)


def _codebook_full(codebook, dec_W, dec_b_row, F):
    nblk = 8
    rb = M // nblk
    return pl.pallas_call(
        _cbfull_body,
        grid=(nblk,),
        in_specs=[pl.BlockSpec((rb, D), lambda i: (i, 0)),
                  pl.BlockSpec((D, F), lambda i: (0, 0)),
                  pl.BlockSpec((1, F), lambda i: (0, 0))],
        out_specs=[pl.BlockSpec((rb, F), lambda i: (i, 0)),
                   pl.BlockSpec((1, rb), lambda i: (0, i))],
        out_shape=[jax.ShapeDtypeStruct((M, F), jnp.float32),
                   jax.ShapeDtypeStruct((1, M), jnp.float32)],
    )(codebook, dec_W, dec_b_row)


def _make_patchify_sc(B, C, H, W, F, row0, nrow):
    wp = W // P
    hp = H // P
    info = plsc.get_sparse_core_info()
    NW = info.num_cores * info.num_subcores
    rows_per_w = nrow // NW
    mesh = plsc.VectorSubcoreMesh(core_axis_name="c", subcore_axis_name="s")

    @functools.partial(
        pl.kernel, mesh=mesh,
        out_type=jax.ShapeDtypeStruct((nrow * wp, F), jnp.float32),
        scratch_types=[pltpu.VMEM((2, C, P, W), jnp.float32),
                       pltpu.VMEM((2, wp, F), jnp.float32),
                       pltpu.SemaphoreType.DMA,
                       pltpu.SemaphoreType.DMA,
                       pltpu.SemaphoreType.DMA,
                       pltpu.SemaphoreType.DMA],
    )
    def patchify(x_hbm, patches_hbm, slab_v, patch_v, s0, s1, w0, w1):
        wid = lax.axis_index("s") * info.num_cores + lax.axis_index("c")
        rsem = (s0, s1)
        wsem = (w0, w1)

        def fire_reads(k, buf):
            row = row0 + wid * rows_per_w + k
            b = row // hp
            i = row % hp
            return [pltpu.async_copy(x_hbm.at[b, c, pl.ds(i * P, P)],
                                     slab_v.at[buf, c], rsem[buf])
                    for c in range(C)]

        reads = fire_reads(0, 0)
        writes = [None, None]
        for k in range(rows_per_w):
            cur = k & 1
            nxt = 1 - cur
            if k + 1 < rows_per_w:
                nreads = fire_reads(k + 1, nxt)
            for d in reads:
                d.wait()
            if writes[cur] is not None:
                writes[cur].wait()

            def rearrange(jj, _):
                for u in range(2):
                    j = jj * 2 + u
                    for c in range(C):
                        for pr in range(P):
                            patch_v[cur, j, pl.ds((c * P + pr) * P, P)] = (
                                slab_v[cur, c, pr, pl.ds(j * P, P)])
                return 0

            lax.fori_loop(0, wp // 2, rearrange, 0)
            lrow = wid * rows_per_w + k
            writes[cur] = pltpu.async_copy(
                patch_v.at[cur], patches_hbm.at[pl.ds(lrow * wp, wp)],
                wsem[cur])
            if k + 1 < rows_per_w:
                reads = nreads
        for d in writes:
            if d is not None:
                d.wait()

    return patchify


def _assign_body(nblk, p_ref, enc_w_ref, enc_b_ref, cbt_ref, cbn2_ref,
                 ind_ref, vq_ref):
    s = pl.program_id(0)
    lat = jnp.dot(p_ref[...], enc_w_ref[...],
                  preferred_element_type=jnp.float32) + enc_b_ref[...]
    scores = jnp.dot(lat, cbt_ref[...], preferred_element_type=jnp.float32)
    best = jnp.max(scores, axis=1)
    ind = jnp.argmax(scores, axis=1).astype(jnp.int32)

    onehot = (jax.lax.broadcasted_iota(jnp.int32, (TBLK, M), 1)
              == ind[:, None]).astype(jnp.float32)
    qn2 = jnp.sum(onehot * cbn2_ref[...], axis=1)
    ln2 = jnp.sum(lat * lat, axis=1)
    cos = best / jnp.maximum(jnp.sqrt(ln2) * jnp.sqrt(qn2), 1e-8)
    vq_blk = jnp.sum(1.0 - cos)

    ind_ref[...] = ind.reshape(1, 1, TBLK)

    @pl.when(s == 0)
    def _():
        vq_ref[0, 0] = 0.0

    vq_ref[0, 0] += vq_blk


def _assign(patches, enc_W, enc_b_row, codebook_T, cbn2, N, F):
    nblk = N // TBLK
    ind, vq = pl.pallas_call(
        functools.partial(_assign_body, nblk),
        grid=(nblk,),
        in_specs=[pl.BlockSpec((TBLK, F), lambda s: (s, 0)),
                  pl.BlockSpec((F, D), lambda s: (0, 0)),
                  pl.BlockSpec((1, D), lambda s: (0, 0)),
                  pl.BlockSpec((D, M), lambda s: (0, 0)),
                  pl.BlockSpec((1, M), lambda s: (0, 0))],
        out_specs=[pl.BlockSpec((1, 1, TBLK), lambda s: (s, 0, 0)),
                   pl.BlockSpec((1, 1), lambda s: (0, 0),
                                memory_space=pltpu.SMEM)],
        out_shape=[jax.ShapeDtypeStruct((nblk, 1, TBLK), jnp.int32),
                   jax.ShapeDtypeStruct((1, 1), jnp.float32)],
    )(patches, enc_W, enc_b_row, codebook_T, cbn2)
    return ind.reshape(N), vq


def _make_decode_sc(B, C, H, W, F):
    wp = W // P
    hp = H // P
    nrow = B * hp
    info = plsc.get_sparse_core_info()
    NW = info.num_cores * info.num_subcores
    rows_per_w = nrow // NW
    mesh = plsc.VectorSubcoreMesh(core_axis_name="c", subcore_axis_name="s")

    @functools.partial(
        pl.kernel, mesh=mesh,
        out_type=jax.ShapeDtypeStruct((B, C, H, W), jnp.float32),
        scratch_types=[pltpu.VMEM((rows_per_w * wp,), jnp.int32),
                       pltpu.VMEM((2, wp, F), jnp.float32),
                       pltpu.VMEM((2, C, P, W), jnp.float32),
                       pltpu.SemaphoreType.DMA,
                       pltpu.SemaphoreType.DMA,
                       pltpu.SemaphoreType.DMA,
                       pltpu.SemaphoreType.DMA],
    )
    def decode(cbfull_hbm, ind_hbm, out_hbm, idx_v, rows_v, slab_v,
               g0, g1, w0, w1):
        wid = lax.axis_index("s") * info.num_cores + lax.axis_index("c")
        gsem = (g0, g1)
        wsem = (w0, w1)

        pltpu.sync_copy(ind_hbm.at[pl.ds(wid * rows_per_w * wp,
                                         rows_per_w * wp)], idx_v)

        def fire_gather(k, buf):
            return pltpu.async_copy(cbfull_hbm.at[idx_v.at[pl.ds(k * wp, wp)]],
                                    rows_v.at[buf], gsem[buf])

        gd = fire_gather(0, 0)
        writes = [None, None]
        for k in range(rows_per_w):
            cur = k & 1
            nxt = 1 - cur
            if k + 1 < rows_per_w:
                ngd = fire_gather(k + 1, nxt)
            gd.wait()
            if writes[cur] is not None:
                for d in writes[cur]:
                    d.wait()

            def rearrange(jj, _):
                for u in range(2):
                    j = jj * 2 + u
                    for c in range(C):
                        for pr in range(P):
                            slab_v[cur, c, pr, pl.ds(j * P, P)] = (
                                rows_v[cur, j, pl.ds((c * P + pr) * P, P)])
                return 0

            lax.fori_loop(0, wp // 2, rearrange, 0)
            row = wid * rows_per_w + k
            b = row // hp
            i = row % hp
            writes[cur] = [pltpu.async_copy(
                slab_v.at[cur, c], out_hbm.at[b, c, pl.ds(i * P, P)],
                wsem[cur]) for c in range(C)]
            if k + 1 < rows_per_w:
                gd = ngd
        for ds_ in writes:
            if ds_ is not None:
                for d in ds_:
                    d.wait()

    return decode


def _rec_body(nb, s_ref, t_ref, vq_ref, rec_ref, loss_ref):
    b = pl.program_id(0)
    diff = s_ref[...] - t_ref[...]
    blk = jnp.sum(diff * diff)

    @pl.when(b == 0)
    def _():
        rec_ref[0, 0] = 0.0

    rec_ref[0, 0] += blk

    @pl.when(b == nb - 1)
    def _():
        loss_ref[0, 0] = rec_ref[0, 0] + 0.001 * vq_ref[0, 0]


def _rec_loss(sample, target, vq, B, C, H, W):
    return pl.pallas_call(
        functools.partial(_rec_body, B),
        grid=(B,),
        in_specs=[pl.BlockSpec((1, C, H, W), lambda b: (b, 0, 0, 0)),
                  pl.BlockSpec((1, C, H, W), lambda b: (b, 0, 0, 0)),
                  pl.BlockSpec((1, 1), lambda b: (0, 0),
                               memory_space=pltpu.SMEM)],
        out_specs=[pl.BlockSpec((1, 1), lambda b: (0, 0),
                                memory_space=pltpu.SMEM),
                   pl.BlockSpec((1, 1), lambda b: (0, 0),
                                memory_space=pltpu.SMEM)],
        out_shape=[jax.ShapeDtypeStruct((1, 1), jnp.float32),
                   jax.ShapeDtypeStruct((1, 1), jnp.float32)],
    )(sample, target, vq)


def kernel(input, target, enc_W, enc_b, codebook, dec_W, dec_b):
    B, C, H, W = input.shape
    F = C * P * P
    hp = H // P
    nrow = B * hp
    half = nrow // 2

    cbfull, cbn2 = _codebook_full(codebook, dec_W, dec_b.reshape(1, F), F)
    enc_b_row = enc_b.reshape(1, D)
    cbT = codebook.T

    patches_a = _make_patchify_sc(B, C, H, W, F, 0, half)(input)
    patches_b = _make_patchify_sc(B, C, H, W, F, half, half)(input)
    N2 = half * (W // P)
    ind_a, vq_a = _assign(patches_a, enc_W, enc_b_row, cbT, cbn2, N2, F)
    ind_b, vq_b = _assign(patches_b, enc_W, enc_b_row, cbT, cbn2, N2, F)
    vq = vq_a + vq_b
    ind = jnp.concatenate([ind_a, ind_b])
    sample = _make_decode_sc(B, C, H, W, F)(cbfull, ind)
    rec, loss = _rec_loss(sample, target, vq, B, C, H, W)

    return sample, rec[0, 0], vq[0, 0], loss[0, 0]
